# baseline (speedup 1.0000x reference)
import jax
import jax.numpy as jnp
from jax.experimental import pallas as pl
from jax.experimental.pallas import tpu as pltpu

N = 1000
M = 300
MP = 304
D = 64
NTA = 200
TA = N // NTA
NTB = 40
TB = N // NTB
XB = NTB * MP
NTE = 50
TE = N // NTE
XE = NTE * MP
XT = N * MP
RB = 15200
TC = XT // RB
CNT = float(N * M)
NPAD = float(N * (MP - M))
EPS_BN = 1e-5

_b16 = lambda v: v.astype(jnp.bfloat16).astype(jnp.float32)

_ARB = pltpu.CompilerParams(dimension_semantics=("arbitrary",))


def _finalize(st, g, b, C):
    mean = st[0:1, 0:C] * (1.0 / CNT)
    var = jnp.maximum(st[1:2, 0:C] * (1.0 / CNT) - mean * mean, 0.0)
    s = g * jax.lax.rsqrt(var + EPS_BN)
    return s, b - mean * s


def _body_a(det_ref, emb_ref, keT_ref, rbT_ref, cos_o, iou_o, st0_o):
    t = pl.program_id(0)

    @pl.when(t == 0)
    def _():
        st0_o[...] = jnp.zeros_like(st0_o)

    emb = emb_ref[...]
    nrm = jnp.sqrt(jnp.sum(emb * emb, axis=1, keepdims=True))
    qe = emb / jnp.maximum(nrm, 1e-8)
    kt = keT_ref[...]
    kn = jnp.sqrt(jnp.sum(kt * kt, axis=0, keepdims=True))
    ktn = kt / jnp.maximum(kn, 1e-8)
    cos = jnp.dot(qe.astype(jnp.bfloat16), ktn.astype(jnp.bfloat16),
                  preferred_element_type=jnp.float32)

    db = det_ref[...]
    ax1, ay1, ax2, ay2 = (db[:, 0:1], db[:, 1:2], db[:, 2:3], db[:, 3:4])
    rb = rbT_ref[...]
    bx1, by1, bx2, by2 = (rb[0:1, :], rb[1:2, :], rb[2:3, :], rb[3:4, :])
    w = jnp.maximum(jnp.minimum(ax2, bx2) - jnp.maximum(ax1, bx1), 0.0)
    h = jnp.maximum(jnp.minimum(ay2, by2) - jnp.maximum(ay1, by1), 0.0)
    inter = w * h
    area_a = (ax2 - ax1) * (ay2 - ay1)
    area_b = (bx2 - bx1) * (by2 - by1)
    iou = inter / (area_a + area_b - inter + 1e-9)

    cos_o[...] = cos
    iou_o[...] = iou
    st0_o[0:1, 0:1] += jnp.sum(cos).reshape(1, 1)
    st0_o[0:1, 1:2] += jnp.sum(iou).reshape(1, 1)
    st0_o[1:2, 0:1] += jnp.sum(cos * cos).reshape(1, 1)
    st0_o[1:2, 1:2] += jnp.sum(iou * iou).reshape(1, 1)


def _body_b(cos_ref, iou_ref, st0_ref, g0_ref, b0_ref, W1_ref, b1_ref,
            a1_o, st1_o):
    t = pl.program_id(0)

    @pl.when(t == 0)
    def _():
        st1_o[...] = jnp.zeros_like(st1_o)

    s0, t0 = _finalize(st0_ref, g0_ref[...], b0_ref[...], 2)
    W1 = W1_ref[...]
    A1 = _b16(W1[0:1, :])
    B1 = _b16(W1[1:2, :])
    b1 = b1_ref[...]
    fc = _b16(cos_ref[...] * s0[0:1, 0:1] + t0[0:1, 0:1])
    fi = _b16(iou_ref[...] * s0[0:1, 1:2] + t0[0:1, 1:2])
    a3 = (fc[:, :, None] * A1[None, :, :] + fi[:, :, None] * B1[None, :, :]
          + b1[None, :, :])
    a1 = jnp.maximum(a3, 0.0).reshape(XB, 32)
    a1_o[...] = a1
    st1_o[0:1, 0:32] += jnp.sum(a1, axis=0, keepdims=True)
    st1_o[1:2, 0:32] += jnp.sum(a1 * a1, axis=0, keepdims=True)

    @pl.when(t == TB - 1)
    def _():
        fcp = _b16(jnp.zeros((1, 1), jnp.float32) * s0[0:1, 0:1]
                   + t0[0:1, 0:1])
        fip = _b16(jnp.zeros((1, 1), jnp.float32) * s0[0:1, 1:2]
                   + t0[0:1, 1:2])
        c1 = jnp.maximum(fcp * A1 + fip * B1 + b1, 0.0)
        st1_o[0:1, 0:32] += -NPAD * c1
        st1_o[1:2, 0:32] += -NPAD * (c1 * c1)
        st1_o[2:3, 0:32] = c1


def _mlp_stage(a_ref, stp_ref, g_ref, b_ref, W_ref, bias_ref, a_o, st_o,
               Cin, Cout, relu, last_t):
    t = pl.program_id(0)

    @pl.when(t == 0)
    def _():
        st_o[...] = jnp.zeros_like(st_o)

    s, tt = _finalize(stp_ref, g_ref[...], b_ref[...], Cin)
    Wb = W_ref[...].astype(jnp.bfloat16)
    bias = bias_ref[...]
    h = jnp.dot((a_ref[...] * s + tt).astype(jnp.bfloat16), Wb,
                preferred_element_type=jnp.float32) + bias
    a = jnp.maximum(h, 0.0) if relu else h
    a_o[...] = a
    st_o[0:1, 0:Cout] += jnp.sum(a, axis=0, keepdims=True)
    st_o[1:2, 0:Cout] += jnp.sum(a * a, axis=0, keepdims=True)

    @pl.when(t == last_t)
    def _():
        cp = stp_ref[2:3, 0:Cin]
        c = jnp.dot((cp * s + tt).astype(jnp.bfloat16), Wb,
                    preferred_element_type=jnp.float32) + bias
        c = jnp.maximum(c, 0.0)
        st_o[0:1, 0:Cout] += -NPAD * c
        st_o[1:2, 0:Cout] += -NPAD * (c * c)
        st_o[2:3, 0:Cout] = c


def _body_c(a_ref, stp_ref, g_ref, b_ref, W_ref, bias_ref, a_o, st_o):
    _mlp_stage(a_ref, stp_ref, g_ref, b_ref, W_ref, bias_ref, a_o, st_o,
               32, 32, True, TC - 1)


def _body_d(a_ref, stp_ref, g_ref, b_ref, W_ref, bias_ref, st_o):
    t = pl.program_id(0)

    @pl.when(t == 0)
    def _():
        st_o[...] = jnp.zeros_like(st_o)

    s, tt = _finalize(stp_ref, g_ref[...], b_ref[...], 32)
    Wb = W_ref[...].astype(jnp.bfloat16)
    bias = bias_ref[...]
    h = jnp.dot((a_ref[...] * s + tt).astype(jnp.bfloat16), Wb,
                preferred_element_type=jnp.float32) + bias
    a = jnp.maximum(h, 0.0)
    st_o[0:1, 0:64] += jnp.sum(a, axis=0, keepdims=True)
    st_o[1:2, 0:64] += jnp.sum(a * a, axis=0, keepdims=True)

    @pl.when(t == TC - 1)
    def _():
        cp = stp_ref[2:3, 0:32]
        c = jnp.maximum(jnp.dot((cp * s + tt).astype(jnp.bfloat16), Wb,
                                preferred_element_type=jnp.float32) + bias,
                        0.0)
        st_o[0:1, 0:64] += -NPAD * c
        st_o[1:2, 0:64] += -NPAD * (c * c)
        st_o[2:3, 0:64] = c


def _body_e(a_ref, st2_ref, st3_ref, g2_ref, b2_ref, W3_ref, b3b_ref,
            g3_ref, b3_ref, W4_ref, b4_ref, out_ref):
    s2, tt2 = _finalize(st2_ref, g2_ref[...], b2_ref[...], 32)
    a3 = jnp.maximum(
        jnp.dot((a_ref[...] * s2 + tt2).astype(jnp.bfloat16),
                W3_ref[...].astype(jnp.bfloat16),
                preferred_element_type=jnp.float32) + b3b_ref[...], 0.0)
    s3, tt3 = _finalize(st3_ref, g3_ref[...], b3_ref[...], 64)
    o = jnp.dot((a3 * s3 + tt3).astype(jnp.bfloat16),
                W4_ref[...].astype(jnp.bfloat16),
                preferred_element_type=jnp.float32) + b4_ref[...]
    out_ref[...] = o.reshape(NTE, MP, 64)[:, :M, :]


def _spec(shape, imap):
    return pl.BlockSpec(shape, imap)


@jax.jit
def kernel(det_boxes, id_embeds, ref_boxes, ref_embeds,
           bn0_g, bn0_b, W1, b1, bn1_g, bn1_b, W2, b2,
           bn2_g, bn2_b, W3, b3, bn3_g, bn3_b, W4, b4):
    f32 = jnp.float32
    keT = jnp.zeros((D, MP), f32).at[:, :M].set(ref_embeds.T)
    rbT = jnp.zeros((4, MP), f32).at[:, :M].set(ref_boxes.T)
    r1 = lambda v: v.reshape(1, -1)
    c0 = lambda s: _spec(s, lambda t: (0, 0))
    rowt = lambda s: _spec(s, lambda t: (t, 0))

    cos, iou, st0 = pl.pallas_call(
        _body_a, grid=(TA,),
        in_specs=[rowt((NTA, 4)), rowt((NTA, D)), c0((D, MP)), c0((4, MP))],
        out_specs=(rowt((NTA, MP)), rowt((NTA, MP)), c0((2, 128))),
        out_shape=(jax.ShapeDtypeStruct((N, MP), f32),
                   jax.ShapeDtypeStruct((N, MP), f32),
                   jax.ShapeDtypeStruct((2, 128), f32)),
        compiler_params=_ARB,
    )(det_boxes, id_embeds, keT, rbT)

    a1, st1 = pl.pallas_call(
        _body_b, grid=(TB,),
        in_specs=[rowt((NTB, MP)), rowt((NTB, MP)), c0((2, 128)),
                  c0((1, 2)), c0((1, 2)), c0((2, 32)), c0((1, 32))],
        out_specs=(rowt((XB, 32)), c0((3, 128))),
        out_shape=(jax.ShapeDtypeStruct((XT, 32), f32),
                   jax.ShapeDtypeStruct((3, 128), f32)),
        compiler_params=_ARB,
    )(cos, iou, st0, r1(bn0_g), r1(bn0_b), W1, r1(b1))

    a2, st2 = pl.pallas_call(
        _body_c, grid=(TC,),
        in_specs=[rowt((RB, 32)), c0((3, 128)),
                  c0((1, 32)), c0((1, 32)), c0((32, 32)), c0((1, 32))],
        out_specs=(rowt((RB, 32)), c0((3, 128))),
        out_shape=(jax.ShapeDtypeStruct((XT, 32), f32),
                   jax.ShapeDtypeStruct((3, 128), f32)),
        compiler_params=_ARB,
    )(a1, st1, r1(bn1_g), r1(bn1_b), W2, r1(b2))

    st3 = pl.pallas_call(
        _body_d, grid=(TC,),
        in_specs=[rowt((RB, 32)), c0((3, 128)),
                  c0((1, 32)), c0((1, 32)), c0((32, 64)), c0((1, 64))],
        out_specs=c0((3, 128)),
        out_shape=jax.ShapeDtypeStruct((3, 128), f32),
        compiler_params=_ARB,
    )(a2, st2, r1(bn2_g), r1(bn2_b), W3, r1(b3))

    out = pl.pallas_call(
        _body_e, grid=(TE,),
        in_specs=[rowt((XE, 32)), c0((3, 128)), c0((3, 128)),
                  c0((1, 32)), c0((1, 32)), c0((32, 64)), c0((1, 64)),
                  c0((1, 64)), c0((1, 64)), c0((64, 64)), c0((1, 64))],
        out_specs=pl.BlockSpec((NTE, M, 64), lambda t: (t, 0, 0)),
        out_shape=jax.ShapeDtypeStruct((N, M, 64), f32),
        compiler_params=_ARB,
    )(a2, st2, st3, r1(bn2_g), r1(bn2_b), W3, r1(b3),
      r1(bn3_g), r1(bn3_b), W4, r1(b4))
    return out


# speedup vs baseline: 1.0121x; 1.0121x over previous
import jax
import jax.numpy as jnp
from jax.experimental import pallas as pl
from jax.experimental.pallas import tpu as pltpu

N = 1000
M = 300
MP = 304
D = 64
NTA = 200
TA = N // NTA
NTB = 40
TB = N // NTB
XB = NTB * MP
NTE = 50
TE = N // NTE
XE = NTE * MP
XT = N * MP
RB = 15200
TC = XT // RB
CNT = float(N * M)
NPAD = float(N * (MP - M))
EPS_BN = 1e-5

_b16 = lambda v: v.astype(jnp.bfloat16).astype(jnp.float32)

_ARB = pltpu.CompilerParams(dimension_semantics=("arbitrary",))


def _finalize(st, g, b, C):
    mean = st[0:1, 0:C] * (1.0 / CNT)
    var = jnp.maximum(st[1:2, 0:C] * (1.0 / CNT) - mean * mean, 0.0)
    s = g * jax.lax.rsqrt(var + EPS_BN)
    return s, b - mean * s


def _body_a(det_ref, emb_ref, keT_ref, rbT_ref, cos_o, iou_o, st0_o):
    t = pl.program_id(0)

    @pl.when(t == 0)
    def _():
        st0_o[...] = jnp.zeros_like(st0_o)

    emb = emb_ref[...]
    nrm = jnp.sqrt(jnp.sum(emb * emb, axis=1, keepdims=True))
    qe = emb / jnp.maximum(nrm, 1e-8)
    kt = keT_ref[...]
    kn = jnp.sqrt(jnp.sum(kt * kt, axis=0, keepdims=True))
    ktn = kt / jnp.maximum(kn, 1e-8)
    cos = jnp.dot(qe.astype(jnp.bfloat16), ktn.astype(jnp.bfloat16),
                  preferred_element_type=jnp.float32)

    db = det_ref[...]
    ax1, ay1, ax2, ay2 = (db[:, 0:1], db[:, 1:2], db[:, 2:3], db[:, 3:4])
    rb = rbT_ref[...]
    bx1, by1, bx2, by2 = (rb[0:1, :], rb[1:2, :], rb[2:3, :], rb[3:4, :])
    w = jnp.maximum(jnp.minimum(ax2, bx2) - jnp.maximum(ax1, bx1), 0.0)
    h = jnp.maximum(jnp.minimum(ay2, by2) - jnp.maximum(ay1, by1), 0.0)
    inter = w * h
    area_a = (ax2 - ax1) * (ay2 - ay1)
    area_b = (bx2 - bx1) * (by2 - by1)
    iou = inter / (area_a + area_b - inter + 1e-9)

    cos_o[...] = cos
    iou_o[...] = iou
    st0_o[0:1, 0:1] += jnp.sum(cos).reshape(1, 1)
    st0_o[0:1, 1:2] += jnp.sum(iou).reshape(1, 1)
    st0_o[1:2, 0:1] += jnp.sum(cos * cos).reshape(1, 1)
    st0_o[1:2, 1:2] += jnp.sum(iou * iou).reshape(1, 1)


def _body_b(cos_ref, iou_ref, st0_ref, g0_ref, b0_ref, W1_ref, b1_ref,
            a1_o, st1_o):
    t = pl.program_id(0)

    @pl.when(t == 0)
    def _():
        st1_o[...] = jnp.zeros_like(st1_o)

    s0, t0 = _finalize(st0_ref, g0_ref[...], b0_ref[...], 2)
    W1 = W1_ref[...]
    A1 = _b16(W1[0:1, :])
    B1 = _b16(W1[1:2, :])
    b1 = b1_ref[...]
    fc = _b16(cos_ref[...] * s0[0:1, 0:1] + t0[0:1, 0:1])
    fi = _b16(iou_ref[...] * s0[0:1, 1:2] + t0[0:1, 1:2])
    feat = jnp.stack([fc, fi], axis=-1).reshape(XB, 2)
    a1 = jnp.maximum(
        jnp.dot(feat.astype(jnp.bfloat16), W1.astype(jnp.bfloat16),
                preferred_element_type=jnp.float32) + b1, 0.0)
    a1_o[...] = a1
    st1_o[0:1, 0:32] += jnp.sum(a1, axis=0, keepdims=True)
    st1_o[1:2, 0:32] += jnp.sum(a1 * a1, axis=0, keepdims=True)

    @pl.when(t == TB - 1)
    def _():
        fcp = _b16(jnp.zeros((1, 1), jnp.float32) * s0[0:1, 0:1]
                   + t0[0:1, 0:1])
        fip = _b16(jnp.zeros((1, 1), jnp.float32) * s0[0:1, 1:2]
                   + t0[0:1, 1:2])
        c1 = jnp.maximum(fcp * A1 + fip * B1 + b1, 0.0)
        st1_o[0:1, 0:32] += -NPAD * c1
        st1_o[1:2, 0:32] += -NPAD * (c1 * c1)
        st1_o[2:3, 0:32] = c1


def _mlp_stage(a_ref, stp_ref, g_ref, b_ref, W_ref, bias_ref, a_o, st_o,
               Cin, Cout, relu, last_t):
    t = pl.program_id(0)

    @pl.when(t == 0)
    def _():
        st_o[...] = jnp.zeros_like(st_o)

    s, tt = _finalize(stp_ref, g_ref[...], b_ref[...], Cin)
    Wb = W_ref[...].astype(jnp.bfloat16)
    bias = bias_ref[...]
    h = jnp.dot((a_ref[...] * s + tt).astype(jnp.bfloat16), Wb,
                preferred_element_type=jnp.float32) + bias
    a = jnp.maximum(h, 0.0) if relu else h
    a_o[...] = a
    st_o[0:1, 0:Cout] += jnp.sum(a, axis=0, keepdims=True)
    st_o[1:2, 0:Cout] += jnp.sum(a * a, axis=0, keepdims=True)

    @pl.when(t == last_t)
    def _():
        cp = stp_ref[2:3, 0:Cin]
        c = jnp.dot((cp * s + tt).astype(jnp.bfloat16), Wb,
                    preferred_element_type=jnp.float32) + bias
        c = jnp.maximum(c, 0.0)
        st_o[0:1, 0:Cout] += -NPAD * c
        st_o[1:2, 0:Cout] += -NPAD * (c * c)
        st_o[2:3, 0:Cout] = c


def _body_c(a_ref, stp_ref, g_ref, b_ref, W_ref, bias_ref, a_o, st_o):
    _mlp_stage(a_ref, stp_ref, g_ref, b_ref, W_ref, bias_ref, a_o, st_o,
               32, 32, True, TC - 1)


def _body_d(a_ref, stp_ref, g_ref, b_ref, W_ref, bias_ref, st_o):
    t = pl.program_id(0)

    @pl.when(t == 0)
    def _():
        st_o[...] = jnp.zeros_like(st_o)

    s, tt = _finalize(stp_ref, g_ref[...], b_ref[...], 32)
    Wb = W_ref[...].astype(jnp.bfloat16)
    bias = bias_ref[...]
    h = jnp.dot((a_ref[...] * s + tt).astype(jnp.bfloat16), Wb,
                preferred_element_type=jnp.float32) + bias
    a = jnp.maximum(h, 0.0)
    st_o[0:1, 0:64] += jnp.sum(a, axis=0, keepdims=True)
    st_o[1:2, 0:64] += jnp.sum(a * a, axis=0, keepdims=True)

    @pl.when(t == TC - 1)
    def _():
        cp = stp_ref[2:3, 0:32]
        c = jnp.maximum(jnp.dot((cp * s + tt).astype(jnp.bfloat16), Wb,
                                preferred_element_type=jnp.float32) + bias,
                        0.0)
        st_o[0:1, 0:64] += -NPAD * c
        st_o[1:2, 0:64] += -NPAD * (c * c)
        st_o[2:3, 0:64] = c


def _body_e(a_ref, st2_ref, st3_ref, g2_ref, b2_ref, W3_ref, b3b_ref,
            g3_ref, b3_ref, W4_ref, b4_ref, out_ref):
    s2, tt2 = _finalize(st2_ref, g2_ref[...], b2_ref[...], 32)
    a3 = jnp.maximum(
        jnp.dot((a_ref[...] * s2 + tt2).astype(jnp.bfloat16),
                W3_ref[...].astype(jnp.bfloat16),
                preferred_element_type=jnp.float32) + b3b_ref[...], 0.0)
    s3, tt3 = _finalize(st3_ref, g3_ref[...], b3_ref[...], 64)
    o = jnp.dot((a3 * s3 + tt3).astype(jnp.bfloat16),
                W4_ref[...].astype(jnp.bfloat16),
                preferred_element_type=jnp.float32) + b4_ref[...]
    out_ref[...] = o.reshape(NTE, MP, 64)[:, :M, :]


def _spec(shape, imap):
    return pl.BlockSpec(shape, imap)


@jax.jit
def kernel(det_boxes, id_embeds, ref_boxes, ref_embeds,
           bn0_g, bn0_b, W1, b1, bn1_g, bn1_b, W2, b2,
           bn2_g, bn2_b, W3, b3, bn3_g, bn3_b, W4, b4):
    f32 = jnp.float32
    keT = jnp.zeros((D, MP), f32).at[:, :M].set(ref_embeds.T)
    rbT = jnp.zeros((4, MP), f32).at[:, :M].set(ref_boxes.T)
    r1 = lambda v: v.reshape(1, -1)
    c0 = lambda s: _spec(s, lambda t: (0, 0))
    rowt = lambda s: _spec(s, lambda t: (t, 0))

    cos, iou, st0 = pl.pallas_call(
        _body_a, grid=(TA,),
        in_specs=[rowt((NTA, 4)), rowt((NTA, D)), c0((D, MP)), c0((4, MP))],
        out_specs=(rowt((NTA, MP)), rowt((NTA, MP)), c0((2, 128))),
        out_shape=(jax.ShapeDtypeStruct((N, MP), f32),
                   jax.ShapeDtypeStruct((N, MP), f32),
                   jax.ShapeDtypeStruct((2, 128), f32)),
        compiler_params=_ARB,
    )(det_boxes, id_embeds, keT, rbT)

    a1, st1 = pl.pallas_call(
        _body_b, grid=(TB,),
        in_specs=[rowt((NTB, MP)), rowt((NTB, MP)), c0((2, 128)),
                  c0((1, 2)), c0((1, 2)), c0((2, 32)), c0((1, 32))],
        out_specs=(rowt((XB, 32)), c0((3, 128))),
        out_shape=(jax.ShapeDtypeStruct((XT, 32), f32),
                   jax.ShapeDtypeStruct((3, 128), f32)),
        compiler_params=_ARB,
    )(cos, iou, st0, r1(bn0_g), r1(bn0_b), W1, r1(b1))

    a2, st2 = pl.pallas_call(
        _body_c, grid=(TC,),
        in_specs=[rowt((RB, 32)), c0((3, 128)),
                  c0((1, 32)), c0((1, 32)), c0((32, 32)), c0((1, 32))],
        out_specs=(rowt((RB, 32)), c0((3, 128))),
        out_shape=(jax.ShapeDtypeStruct((XT, 32), f32),
                   jax.ShapeDtypeStruct((3, 128), f32)),
        compiler_params=_ARB,
    )(a1, st1, r1(bn1_g), r1(bn1_b), W2, r1(b2))

    st3 = pl.pallas_call(
        _body_d, grid=(TC,),
        in_specs=[rowt((RB, 32)), c0((3, 128)),
                  c0((1, 32)), c0((1, 32)), c0((32, 64)), c0((1, 64))],
        out_specs=c0((3, 128)),
        out_shape=jax.ShapeDtypeStruct((3, 128), f32),
        compiler_params=_ARB,
    )(a2, st2, r1(bn2_g), r1(bn2_b), W3, r1(b3))

    out = pl.pallas_call(
        _body_e, grid=(TE,),
        in_specs=[rowt((XE, 32)), c0((3, 128)), c0((3, 128)),
                  c0((1, 32)), c0((1, 32)), c0((32, 64)), c0((1, 64)),
                  c0((1, 64)), c0((1, 64)), c0((64, 64)), c0((1, 64))],
        out_specs=pl.BlockSpec((NTE, M, 64), lambda t: (t, 0, 0)),
        out_shape=jax.ShapeDtypeStruct((N, M, 64), f32),
        compiler_params=_ARB,
    )(a2, st2, st3, r1(bn2_g), r1(bn2_b), W3, r1(b3),
      r1(bn3_g), r1(bn3_b), W4, r1(b4))
    return out
